# Initial kernel scaffold; baseline (speedup 1.0000x reference)
import jax
import jax.numpy as jnp
from jax.experimental import pallas as pl


def kernel(x, router_q, router_k, router_v, router_o, compress_neurons, expand_neurons):
    raise NotImplementedError("write your pallas kernel here")


# speedup vs baseline: 1.3267x; 1.3267x over previous
import functools
import math

import jax
import jax.numpy as jnp
from jax.experimental import pallas as pl

B, S, D = 1, 2048, 768
R, H = 128, 16
DH = R // H
NC, NE = 64, 16
KC, KE = 8, 2

TB = 256
QB = 512


def _topk_softmax_dense(s, k):
    t, n = s.shape
    iota = jax.lax.broadcasted_iota(jnp.int32, (t, n), 1)
    sel = jnp.zeros((t, n), jnp.bool_)
    neg = jnp.float32(-jnp.inf)
    for _ in range(k):
        cur = jnp.where(sel, neg, s)
        m = jnp.max(cur, axis=1, keepdims=True)
        eq = jnp.logical_and(cur == m, jnp.logical_not(sel))
        li = jnp.min(jnp.where(eq, iota, n), axis=1, keepdims=True)
        sel = jnp.logical_or(sel, iota == li)
    mx = jnp.max(s, axis=1, keepdims=True)
    e = jnp.where(sel, jnp.exp(s - mx), 0.0)
    return e / jnp.sum(e, axis=1, keepdims=True)


def _compress_kernel(x_ref, rq_ref, rk_ref, rv_ref, n2_ref, q_ref, k_ref, v_ref):
    xb = x_ref[...]
    proj = jax.lax.dot_general(xb, n2_ref[...], (((1,), (0,)), ((), ())),
                               preferred_element_type=jnp.float32)
    for r_ref, o_ref in ((rq_ref, q_ref), (rk_ref, k_ref), (rv_ref, v_ref)):
        sc = jax.lax.dot_general(xb, r_ref[...], (((1,), (0,)), ((), ())),
                                 preferred_element_type=jnp.float32)
        w = _topk_softmax_dense(sc, KC)
        acc = jnp.zeros((TB, R), jnp.float32)
        for n in range(NC):
            acc = acc + w[:, n:n + 1] * proj[:, n * R:(n + 1) * R]
        o_ref[...] = acc


def _attn_kernel(q_ref, kt_ref, v_ref, o_ref):
    q = q_ref[0]
    kt = kt_ref[0]
    v = v_ref[0]
    sc = jax.lax.dot_general(q, kt, (((1,), (0,)), ((), ())),
                             preferred_element_type=jnp.float32)
    sc = sc * jnp.float32(1.0 / math.sqrt(DH))
    m = jnp.max(sc, axis=1, keepdims=True)
    p = jnp.exp(sc - m)
    denom = jnp.sum(p, axis=1, keepdims=True)
    o = jax.lax.dot_general(p, v, (((1,), (0,)), ((), ())),
                            preferred_element_type=jnp.float32)
    o_ref[0] = o / denom


def _expand_kernel(a_ref, rot_ref, e2_ref, o_ref):
    ab = a_ref[...]
    sc = jax.lax.dot_general(ab, rot_ref[...], (((1,), (0,)), ((), ())),
                             preferred_element_type=jnp.float32)
    w = _topk_softmax_dense(sc, KE)
    proj = jax.lax.dot_general(ab, e2_ref[...], (((1,), (0,)), ((), ())),
                               preferred_element_type=jnp.float32)
    acc = jnp.zeros((TB, D), jnp.float32)
    for n in range(NE):
        acc = acc + w[:, n:n + 1] * proj[:, n * D:(n + 1) * D]
    o_ref[...] = acc


@functools.partial(jax.jit)
def kernel(x, router_q, router_k, router_v, router_o, compress_neurons, expand_neurons):
    xs = x.reshape(S, D)
    n2 = compress_neurons.transpose(1, 0, 2).reshape(D, NC * R)
    e2 = expand_neurons.transpose(1, 0, 2).reshape(R, NE * D)
    rqt = router_q.T
    rkt = router_k.T
    rvt = router_v.T
    rot = router_o.T

    q, k, v = pl.pallas_call(
        _compress_kernel,
        grid=(S // TB,),
        in_specs=[
            pl.BlockSpec((TB, D), lambda i: (i, 0)),
            pl.BlockSpec((D, NC), lambda i: (0, 0)),
            pl.BlockSpec((D, NC), lambda i: (0, 0)),
            pl.BlockSpec((D, NC), lambda i: (0, 0)),
            pl.BlockSpec((D, NC * R), lambda i: (0, 0)),
        ],
        out_specs=[
            pl.BlockSpec((TB, R), lambda i: (i, 0)),
            pl.BlockSpec((TB, R), lambda i: (i, 0)),
            pl.BlockSpec((TB, R), lambda i: (i, 0)),
        ],
        out_shape=[jax.ShapeDtypeStruct((S, R), jnp.float32)] * 3,
    )(xs, rqt, rkt, rvt, n2)

    qh = q.reshape(S, H, DH).transpose(1, 0, 2)
    kht = k.reshape(S, H, DH).transpose(1, 2, 0)
    vh = v.reshape(S, H, DH).transpose(1, 0, 2)

    ao = pl.pallas_call(
        _attn_kernel,
        grid=(H, S // QB),
        in_specs=[
            pl.BlockSpec((1, QB, DH), lambda h, i: (h, i, 0)),
            pl.BlockSpec((1, DH, S), lambda h, i: (h, 0, 0)),
            pl.BlockSpec((1, S, DH), lambda h, i: (h, 0, 0)),
        ],
        out_specs=pl.BlockSpec((1, QB, DH), lambda h, i: (h, i, 0)),
        out_shape=jax.ShapeDtypeStruct((H, S, DH), jnp.float32),
    )(qh, kht, vh)

    a = ao.transpose(1, 0, 2).reshape(S, R)

    out = pl.pallas_call(
        _expand_kernel,
        grid=(S // TB,),
        in_specs=[
            pl.BlockSpec((TB, R), lambda i: (i, 0)),
            pl.BlockSpec((R, NE), lambda i: (0, 0)),
            pl.BlockSpec((R, NE * D), lambda i: (0, 0)),
        ],
        out_specs=pl.BlockSpec((TB, D), lambda i: (i, 0)),
        out_shape=jax.ShapeDtypeStruct((S, D), jnp.float32),
    )(a, rot, e2)

    return out.reshape(B, S, D)
